# Initial kernel scaffold; baseline (speedup 1.0000x reference)
import jax
import jax.numpy as jnp
from jax.experimental import pallas as pl


def kernel(features, edge_index, edge_weights, W1, W2, gn1_gamma, gn1_beta, gn1_alpha, gn2_gamma, gn2_beta, gn2_alpha, ro1_phi_w, ro1_phi_b, ro1_rho_w, ro1_rho_b, ro2_phi_w, ro2_phi_b, ro2_rho_w, ro2_rho_b):
    raise NotImplementedError("write your pallas kernel here")


# speedup vs baseline: 3.0806x; 3.0806x over previous
import functools

import jax
import jax.numpy as jnp
from jax import lax
from jax.experimental import pallas as pl
from jax.experimental.pallas import tpu as pltpu
from jax.experimental.pallas import tpu_sc as plsc

N = 10000
E = 320000
D = 128
H = 128
RO = H // 2
EPS = 1e-05

NC = 2
NS = 16
LN = 16
NW = NC * NS
EW = E // NW
B = 80
NCHUNK = EW // B
RPT = N // NS
ZR = 25
ZB = 5


def _leaky(x):
    return jnp.where(x >= 0, x, 0.01 * x)


def _mesh():
    return plsc.VectorSubcoreMesh(core_axis_name="c", subcore_axis_name="s",
                                  num_cores=NC, num_subcores=NS)


def _sc_degrees(src3, dst3):
    @functools.partial(
        pl.kernel,
        out_type=jax.ShapeDtypeStruct((NC, NS * (RPT // ZR), ZR, D),
                                      jnp.float32),
        mesh=_mesh(),
        scratch_types=[
            pltpu.VMEM((1, B), jnp.int32),
            pltpu.VMEM((B, D), jnp.float32),
            pltpu.VMEM_SHARED((N, D), jnp.float32),
        ],
    )
    def body(src_h, dst_h, out_h, iv, ones_v, dacc):
        c = lax.axis_index("c")
        s = lax.axis_index("s")
        w = c * NS + s
        base = s * RPT

        def fill_zero(i, carry):
            for j in range(D // LN):
                ones_v[i, pl.ds(j * LN, LN)] = jnp.zeros((LN,), jnp.float32)
            return carry

        lax.fori_loop(0, B, fill_zero, 0)
        for t in range(RPT // B):
            pltpu.sync_copy(ones_v, dacc.at[pl.ds(base + t * B, B)])
        pltpu.sync_copy(ones_v.at[pl.ds(0, RPT % B)],
                        dacc.at[pl.ds(base + (RPT // B) * B, RPT % B)])

        for (which_h, lane0) in ((src_h, 0), (dst_h, LN)):
            def fill_ones(i, carry):
                ones_v[i, pl.ds(lane0, LN)] = jnp.full((LN,), 1.0,
                                                       jnp.float32)
                return carry

            lax.fori_loop(0, B, fill_ones, 0)
            if lane0 == 0:
                plsc.subcore_barrier()

            def chunk(k, carry):
                pltpu.sync_copy(which_h.at[w, k], iv)
                pltpu.sync_copy(ones_v, dacc.at[iv.at[0]], add=True)
                return carry

            lax.fori_loop(0, NCHUNK, chunk, 0)

            def fill_zero_blk(i, carry):
                ones_v[i, pl.ds(lane0, LN)] = jnp.zeros((LN,), jnp.float32)
                return carry

            lax.fori_loop(0, B, fill_zero_blk, 0)

        plsc.subcore_barrier()
        for t in range(RPT // ZR):
            pltpu.sync_copy(dacc.at[pl.ds(base + t * ZR, ZR)],
                            out_h.at[c, s * (RPT // ZR) + t])

    return body(src3, dst3)


def _sc_edge_agg(p, src3, dst3, ew3):
    @functools.partial(
        pl.kernel,
        out_type=jax.ShapeDtypeStruct((NC, NS * (RPT // ZR), ZR, D),
                                      jnp.float32),
        mesh=_mesh(),
        scratch_types=[
            pltpu.VMEM((1, B), jnp.int32),
            pltpu.VMEM((1, B), jnp.int32),
            pltpu.VMEM((1, B), jnp.float32),
            pltpu.VMEM((B, D), jnp.float32),
            pltpu.SemaphoreType.DMA,
            pltpu.VMEM_SHARED((N, D), jnp.float32),
        ],
    )
    def body(p_h, src_h, dst_h, ew_h, out_h,
             sv, dv, wv, rows, sem, acc):
        c = lax.axis_index("c")
        s = lax.axis_index("s")
        w = c * NS + s

        base = s * RPT

        def fill_zero(i, carry):
            for j in range(D // LN):
                rows[i, pl.ds(j * LN, LN)] = jnp.zeros((LN,), jnp.float32)
            return carry

        lax.fori_loop(0, B, fill_zero, 0)
        for t in range(RPT // B):
            pltpu.sync_copy(rows, acc.at[pl.ds(base + t * B, B)])
        pltpu.sync_copy(rows.at[pl.ds(0, RPT % B)],
                        acc.at[pl.ds(base + (RPT // B) * B, RPT % B)])
        plsc.subcore_barrier()

        def chunk(k, carry):
            pltpu.sync_copy(src_h.at[w, k], sv)
            pltpu.sync_copy(dst_h.at[w, k], dv)
            pltpu.sync_copy(ew_h.at[w, k], wv)
            pltpu.async_copy(p_h.at[sv.at[0]], rows, sem).wait()

            def scale(g, icarry):
                ewv = wv[0, pl.ds(g * LN, LN)]
                for l in range(LN):
                    i = g * LN + l
                    coef = jnp.broadcast_to(ewv[l], (LN,))
                    for j in range(D // LN):
                        rows[i, pl.ds(j * LN, LN)] = (
                            rows[i, pl.ds(j * LN, LN)] * coef)
                return icarry

            lax.fori_loop(0, B // LN, scale, 0)
            pltpu.sync_copy(rows, acc.at[dv.at[0]], add=True)
            return carry

        lax.fori_loop(0, NCHUNK, chunk, 0)
        plsc.subcore_barrier()
        for t in range(RPT // ZR):
            pltpu.sync_copy(acc.at[pl.ds(base + t * ZR, ZR)],
                            out_h.at[c, s * (RPT // ZR) + t])

    return body(p, src3, dst3, ew3)


def _tc_prep(degp, features):
    def body(deg_ref, x_ref, q_ref, nsd_ref):
        d = deg_ref[0] + deg_ref[1]
        od = d[:, 0:1]
        idg = d[:, 16:17]
        ns = lax.rsqrt(jnp.maximum(od, 1.0))
        nd = lax.rsqrt(jnp.maximum(idg, 1.0))
        q_ref[...] = x_ref[...] * ns
        nsd_ref[...] = jnp.concatenate(
            [ns, nd, jnp.zeros((N, 6), jnp.float32)], axis=1)

    return pl.pallas_call(
        body,
        out_shape=(
            jax.ShapeDtypeStruct((N, D), jnp.float32),
            jax.ShapeDtypeStruct((N, 8), jnp.float32),
        ),
    )(degp, features)


def _dot(a, b):
    return lax.dot_general(a, b, (((1,), (0,)), ((), ())),
                           precision=lax.Precision.HIGHEST,
                           preferred_element_type=jnp.float32)


def _tc_layer(aggp, nsd, W, gamma, beta, alpha, phi_w, phi_b, rho_w, rho_b):
    def body(aggp_ref, nsd_ref, w_ref, gamma_ref, beta_ref, alpha_ref,
             phiw_ref, phib_ref, rhow_ref, rhob_ref,
             q_ref, r_ref, m_ref):
        ns = nsd_ref[:, 0:1]
        nd = nsd_ref[:, 1:2]
        agg = (aggp_ref[0] + aggp_ref[1]) * nd
        z = _dot(agg, w_ref[...])
        alpha_v = alpha_ref[...]
        mu = jnp.mean(z, axis=0, keepdims=True)
        shifted = z - alpha_v * mu
        var = jnp.mean(shifted * shifted, axis=0, keepdims=True)
        hn = gamma_ref[...] * shifted * lax.rsqrt(var + EPS) + beta_ref[...]
        h = _leaky(hn)
        q_ref[...] = h * ns
        phi = _leaky(_dot(h, phiw_ref[...]) + phib_ref[...])
        pooled = jnp.mean(phi, axis=0, keepdims=True)
        r = _dot(pooled, rhow_ref[...]) + rhob_ref[...]
        r_ref[...] = _leaky(r)
        m_ref[...] = _leaky(jnp.mean(h, axis=0, keepdims=True))

    return pl.pallas_call(
        body,
        out_shape=(
            jax.ShapeDtypeStruct((N, H), jnp.float32),
            jax.ShapeDtypeStruct((1, RO), jnp.float32),
            jax.ShapeDtypeStruct((1, H), jnp.float32),
        ),
    )(aggp, nsd, W, gamma, beta, alpha, phi_w, phi_b, rho_w, rho_b)


def kernel(features, edge_index, edge_weights, W1, W2,
           gn1_gamma, gn1_beta, gn1_alpha, gn2_gamma, gn2_beta, gn2_alpha,
           ro1_phi_w, ro1_phi_b, ro1_rho_w, ro1_rho_b,
           ro2_phi_w, ro2_phi_b, ro2_rho_w, ro2_rho_b):
    src2 = edge_index[0].reshape(NW, NCHUNK, B)
    dst2 = edge_index[1].reshape(NW, NCHUNK, B)
    ew2 = edge_weights.reshape(NW, NCHUNK, B)

    src3 = src2.reshape(NW, NCHUNK, 1, B)
    dst3 = dst2.reshape(NW, NCHUNK, 1, B)
    ew3 = ew2.reshape(NW, NCHUNK, 1, B)

    degp = _sc_degrees(src3, dst3)
    degp = degp.reshape(NC, N, D)
    q1, nsd = _tc_prep(degp, features)

    g1g = gn1_gamma.reshape(1, H)
    g1b = gn1_beta.reshape(1, H)
    g1a = gn1_alpha.reshape(1, H)
    g2g = gn2_gamma.reshape(1, H)
    g2b = gn2_beta.reshape(1, H)
    g2a = gn2_alpha.reshape(1, H)

    aggp1 = _sc_edge_agg(q1, src3, dst3, ew3)
    aggp1 = aggp1.reshape(NC, N, D)
    q2, r1, m1 = _tc_layer(aggp1, nsd, W1, g1g, g1b, g1a,
                           ro1_phi_w, ro1_phi_b.reshape(1, H),
                           ro1_rho_w, ro1_rho_b.reshape(1, RO))
    aggp2 = _sc_edge_agg(q2, src3, dst3, ew3)
    aggp2 = aggp2.reshape(NC, N, D)
    _, r2, m2 = _tc_layer(aggp2, nsd, W2, g2g, g2b, g2a,
                          ro2_phi_w, ro2_phi_b.reshape(1, H),
                          ro2_rho_w, ro2_rho_b.reshape(1, RO))

    return jnp.concatenate([r1, m1, r2, m2], axis=1)


# speedup vs baseline: 4.9566x; 1.6090x over previous
import functools

import jax
import jax.numpy as jnp
from jax import lax
from jax.experimental import pallas as pl
from jax.experimental.pallas import tpu as pltpu
from jax.experimental.pallas import tpu_sc as plsc

N = 10000
E = 320000
D = 128
H = 128
RO = H // 2
EPS = 1e-05

NC = 2
NS = 16
LN = 16
NW = NC * NS
EW = E // NW
B = 80
NCHUNK = EW // B
RPT = N // NS
ZR = 25
ZB = 5


def _leaky(x):
    return jnp.where(x >= 0, x, 0.01 * x)


def _mesh():
    return plsc.VectorSubcoreMesh(core_axis_name="c", subcore_axis_name="s",
                                  num_cores=NC, num_subcores=NS)


GD = 25
NB = NCHUNK // GD


def _sc_degrees(src4, dst4):
    @functools.partial(
        pl.kernel,
        out_type=jax.ShapeDtypeStruct((NC, NS * (RPT // ZR), ZR, D),
                                      jnp.float32),
        mesh=_mesh(),
        scratch_types=[
            pltpu.VMEM((GD, B), jnp.int32),
            pltpu.VMEM((B, D), jnp.float32),
            pltpu.SemaphoreType.DMA,
            pltpu.VMEM_SHARED((N, D), jnp.float32),
        ],
    )
    def body(src_h, dst_h, out_h, ibuf, ones_v, dsem, dacc):
        c = lax.axis_index("c")
        s = lax.axis_index("s")
        w = c * NS + s
        base = s * RPT

        def fill_zero(i, carry):
            for j in range(D // LN):
                ones_v[i, pl.ds(j * LN, LN)] = jnp.zeros((LN,), jnp.float32)
            return carry

        lax.fori_loop(0, B, fill_zero, 0)
        for t in range(RPT // B):
            pltpu.sync_copy(ones_v, dacc.at[pl.ds(base + t * B, B)])
        pltpu.sync_copy(ones_v.at[pl.ds(0, RPT % B)],
                        dacc.at[pl.ds(base + (RPT // B) * B, RPT % B)])

        for (which_h, lane0) in ((src_h, 0), (dst_h, LN)):
            def fill_ones(i, carry):
                ones_v[i, pl.ds(lane0, LN)] = jnp.full((LN,), 1.0,
                                                       jnp.float32)
                return carry

            lax.fori_loop(0, B, fill_ones, 0)
            if lane0 == 0:
                plsc.subcore_barrier()

            def batch(g, carry):
                pltpu.sync_copy(which_h.at[w, g], ibuf)
                for j0 in range(0, GD, 5):
                    for j in range(j0, j0 + 5):
                        pltpu.async_copy(ones_v, dacc.at[ibuf.at[j]],
                                         dsem, add=True)
                    for j in range(j0, j0 + 5):
                        pltpu.make_async_copy(ones_v, dacc.at[ibuf.at[j]],
                                              dsem).wait()
                return carry

            lax.fori_loop(0, NB, batch, 0)

            def fill_zero_blk(i, carry):
                ones_v[i, pl.ds(lane0, LN)] = jnp.zeros((LN,), jnp.float32)
                return carry

            if lane0 == 0:
                lax.fori_loop(0, B, fill_zero_blk, 0)

        plsc.subcore_barrier()
        for t in range(RPT // ZR):
            pltpu.sync_copy(dacc.at[pl.ds(base + t * ZR, ZR)],
                            out_h.at[c, s * (RPT // ZR) + t])

    return body(src4, dst4)


def _sc_edge_agg(p, eidx, ew3):
    @functools.partial(
        pl.kernel,
        out_type=jax.ShapeDtypeStruct((NC, NS * (RPT // ZR), ZR, D),
                                      jnp.float32),
        mesh=_mesh(),
        scratch_types=[
            pltpu.VMEM((2, 2, B), jnp.int32),
            pltpu.VMEM((2, 1, B), jnp.float32),
            pltpu.VMEM((2, B, D), jnp.float32),
            pltpu.SemaphoreType.DMA,
            pltpu.SemaphoreType.DMA,
            pltpu.SemaphoreType.DMA,
            pltpu.SemaphoreType.DMA,
            pltpu.VMEM_SHARED((N, D), jnp.float32),
        ],
    )
    def body(p_h, e_h, w_h, out_h, ebuf, wbuf, rows, g0, g1, s0, s1, acc):
        c = lax.axis_index("c")
        s = lax.axis_index("s")
        w = c * NS + s
        base = s * RPT
        gsem = (g0, g1)
        ssem = (s0, s1)

        def fill_zero(i, carry):
            for j in range(D // LN):
                rows[0, i, pl.ds(j * LN, LN)] = jnp.zeros((LN,), jnp.float32)
            return carry

        lax.fori_loop(0, B, fill_zero, 0)
        for t in range(RPT // B):
            pltpu.sync_copy(rows.at[0], acc.at[pl.ds(base + t * B, B)])
        pltpu.sync_copy(rows.at[0, pl.ds(0, RPT % B)],
                        acc.at[pl.ds(base + (RPT // B) * B, RPT % B)])
        plsc.subcore_barrier()

        def scale(b):
            def grp(g, icarry):
                ewv = wbuf[b, 0, pl.ds(g * LN, LN)]
                for l in range(LN):
                    i = g * LN + l
                    coef = jnp.broadcast_to(ewv[l], (LN,))
                    for j in range(D // LN):
                        rows[b, i, pl.ds(j * LN, LN)] = (
                            rows[b, i, pl.ds(j * LN, LN)] * coef)
                return icarry

            lax.fori_loop(0, B // LN, grp, 0)

        def step(k, b):
            pltpu.make_async_copy(p_h.at[ebuf.at[b, 0]], rows.at[b],
                                  gsem[b]).wait()
            scale(b)
            sc = pltpu.async_copy(rows.at[b], acc.at[ebuf.at[b, 1]],
                                  ssem[b], add=True)
            sc.wait()

            @pl.when(k + 2 < NCHUNK)
            def _():
                pltpu.sync_copy(e_h.at[w, k + 2], ebuf.at[b])
                pltpu.sync_copy(w_h.at[w, k + 2], wbuf.at[b])
                pltpu.async_copy(p_h.at[ebuf.at[b, 0]], rows.at[b], gsem[b])

        pltpu.sync_copy(e_h.at[w, 0], ebuf.at[0])
        pltpu.sync_copy(w_h.at[w, 0], wbuf.at[0])
        pltpu.async_copy(p_h.at[ebuf.at[0, 0]], rows.at[0], gsem[0])
        pltpu.sync_copy(e_h.at[w, 1], ebuf.at[1])
        pltpu.sync_copy(w_h.at[w, 1], wbuf.at[1])
        pltpu.async_copy(p_h.at[ebuf.at[1, 0]], rows.at[1], gsem[1])

        def pair(i, carry):
            step(2 * i, 0)
            step(2 * i + 1, 1)
            return carry

        lax.fori_loop(0, NCHUNK // 2, pair, 0)
        if NCHUNK % 2:
            step(NCHUNK - 1, 0)

        plsc.subcore_barrier()
        for t in range(RPT // ZR):
            pltpu.sync_copy(acc.at[pl.ds(base + t * ZR, ZR)],
                            out_h.at[c, s * (RPT // ZR) + t])

    return body(p, eidx, ew3)


def _tc_prep(degp, features):
    def body(deg_ref, x_ref, q_ref, nsd_ref):
        d = deg_ref[0] + deg_ref[1]
        od = d[:, 0:1]
        idg = d[:, 16:17]
        ns = lax.rsqrt(jnp.maximum(od, 1.0))
        nd = lax.rsqrt(jnp.maximum(idg, 1.0))
        q_ref[...] = x_ref[...] * ns
        nsd_ref[...] = jnp.concatenate(
            [ns, nd, jnp.zeros((N, 6), jnp.float32)], axis=1)

    return pl.pallas_call(
        body,
        out_shape=(
            jax.ShapeDtypeStruct((N, D), jnp.float32),
            jax.ShapeDtypeStruct((N, 8), jnp.float32),
        ),
    )(degp, features)


def _dot(a, b):
    return lax.dot_general(a, b, (((1,), (0,)), ((), ())),
                           precision=lax.Precision.HIGHEST,
                           preferred_element_type=jnp.float32)


def _tc_layer(aggp, nsd, W, gamma, beta, alpha, phi_w, phi_b, rho_w, rho_b):
    def body(aggp_ref, nsd_ref, w_ref, gamma_ref, beta_ref, alpha_ref,
             phiw_ref, phib_ref, rhow_ref, rhob_ref,
             q_ref, r_ref, m_ref):
        ns = nsd_ref[:, 0:1]
        nd = nsd_ref[:, 1:2]
        agg = (aggp_ref[0] + aggp_ref[1]) * nd
        z = _dot(agg, w_ref[...])
        alpha_v = alpha_ref[...]
        mu = jnp.mean(z, axis=0, keepdims=True)
        shifted = z - alpha_v * mu
        var = jnp.mean(shifted * shifted, axis=0, keepdims=True)
        hn = gamma_ref[...] * shifted * lax.rsqrt(var + EPS) + beta_ref[...]
        h = _leaky(hn)
        q_ref[...] = h * ns
        phi = _leaky(_dot(h, phiw_ref[...]) + phib_ref[...])
        pooled = jnp.mean(phi, axis=0, keepdims=True)
        r = _dot(pooled, rhow_ref[...]) + rhob_ref[...]
        r_ref[...] = _leaky(r)
        m_ref[...] = _leaky(jnp.mean(h, axis=0, keepdims=True))

    return pl.pallas_call(
        body,
        out_shape=(
            jax.ShapeDtypeStruct((N, H), jnp.float32),
            jax.ShapeDtypeStruct((1, RO), jnp.float32),
            jax.ShapeDtypeStruct((1, H), jnp.float32),
        ),
    )(aggp, nsd, W, gamma, beta, alpha, phi_w, phi_b, rho_w, rho_b)


def kernel(features, edge_index, edge_weights, W1, W2,
           gn1_gamma, gn1_beta, gn1_alpha, gn2_gamma, gn2_beta, gn2_alpha,
           ro1_phi_w, ro1_phi_b, ro1_rho_w, ro1_rho_b,
           ro2_phi_w, ro2_phi_b, ro2_rho_w, ro2_rho_b):
    src2 = edge_index[0].reshape(NW, NCHUNK, B)
    dst2 = edge_index[1].reshape(NW, NCHUNK, B)
    ew2 = edge_weights.reshape(NW, NCHUNK, B)
    eidx = jnp.concatenate(
        [src2[:, :, None, :], dst2[:, :, None, :]],
        axis=2)
    ew3 = ew2.reshape(NW, NCHUNK, 1, B)
    src4 = src2.reshape(NW, NB, GD, B)
    dst4 = dst2.reshape(NW, NB, GD, B)

    degp = _sc_degrees(src4, dst4)
    degp = degp.reshape(NC, N, D)
    q1, nsd = _tc_prep(degp, features)

    g1g = gn1_gamma.reshape(1, H)
    g1b = gn1_beta.reshape(1, H)
    g1a = gn1_alpha.reshape(1, H)
    g2g = gn2_gamma.reshape(1, H)
    g2b = gn2_beta.reshape(1, H)
    g2a = gn2_alpha.reshape(1, H)

    aggp1 = _sc_edge_agg(q1, eidx, ew3)
    aggp1 = aggp1.reshape(NC, N, D)
    q2, r1, m1 = _tc_layer(aggp1, nsd, W1, g1g, g1b, g1a,
                           ro1_phi_w, ro1_phi_b.reshape(1, H),
                           ro1_rho_w, ro1_rho_b.reshape(1, RO))
    aggp2 = _sc_edge_agg(q2, eidx, ew3)
    aggp2 = aggp2.reshape(NC, N, D)
    _, r2, m2 = _tc_layer(aggp2, nsd, W2, g2g, g2b, g2a,
                          ro2_phi_w, ro2_phi_b.reshape(1, H),
                          ro2_rho_w, ro2_rho_b.reshape(1, RO))

    return jnp.concatenate([r1, m1, r2, m2], axis=1)


# speedup vs baseline: 6.6039x; 1.3323x over previous
import functools

import jax
import jax.numpy as jnp
from jax import lax
from jax.experimental import pallas as pl
from jax.experimental.pallas import tpu as pltpu
from jax.experimental.pallas import tpu_sc as plsc

N = 10000
E = 320000
D = 128
H = 128
RO = H // 2
EPS = 1e-05

NC = 2
NS = 16
LN = 16
NW = NC * NS
EW = E // NW
B = 80
NCHUNK = EW // B
RPT = N // NS
ZR = 25
ZB = 5


def _leaky(x):
    return jnp.where(x >= 0, x, 0.01 * x)


def _mesh():
    return plsc.VectorSubcoreMesh(core_axis_name="c", subcore_axis_name="s",
                                  num_cores=NC, num_subcores=NS)


GD = 25
NB = NCHUNK // GD


def _sc_degrees(src4, dst4):
    @functools.partial(
        pl.kernel,
        out_type=jax.ShapeDtypeStruct((NC, NS * (RPT // ZR), ZR, D),
                                      jnp.float32),
        mesh=_mesh(),
        scratch_types=[
            pltpu.VMEM((GD, B), jnp.int32),
            pltpu.VMEM((B, D), jnp.float32),
            pltpu.SemaphoreType.DMA,
            pltpu.VMEM_SHARED((N, D), jnp.float32),
        ],
    )
    def body(src_h, dst_h, out_h, ibuf, ones_v, dsem, dacc):
        c = lax.axis_index("c")
        s = lax.axis_index("s")
        w = c * NS + s
        base = s * RPT

        def fill_zero(i, carry):
            for j in range(D // LN):
                ones_v[i, pl.ds(j * LN, LN)] = jnp.zeros((LN,), jnp.float32)
            return carry

        lax.fori_loop(0, B, fill_zero, 0)
        for t in range(RPT // B):
            pltpu.sync_copy(ones_v, dacc.at[pl.ds(base + t * B, B)])
        pltpu.sync_copy(ones_v.at[pl.ds(0, RPT % B)],
                        dacc.at[pl.ds(base + (RPT // B) * B, RPT % B)])

        for (which_h, lane0) in ((src_h, 0), (dst_h, LN)):
            def fill_ones(i, carry):
                ones_v[i, pl.ds(lane0, LN)] = jnp.full((LN,), 1.0,
                                                       jnp.float32)
                return carry

            lax.fori_loop(0, B, fill_ones, 0)
            if lane0 == 0:
                plsc.subcore_barrier()

            def batch(g, carry):
                pltpu.sync_copy(which_h.at[w, g], ibuf)
                for j0 in range(0, GD, 5):
                    for j in range(j0, j0 + 5):
                        pltpu.async_copy(ones_v, dacc.at[ibuf.at[j]],
                                         dsem, add=True)
                    for j in range(j0, j0 + 5):
                        pltpu.make_async_copy(ones_v, dacc.at[ibuf.at[j]],
                                              dsem).wait()
                return carry

            lax.fori_loop(0, NB, batch, 0)

            def fill_zero_blk(i, carry):
                ones_v[i, pl.ds(lane0, LN)] = jnp.zeros((LN,), jnp.float32)
                return carry

            if lane0 == 0:
                lax.fori_loop(0, B, fill_zero_blk, 0)

        plsc.subcore_barrier()
        for t in range(RPT // ZR):
            pltpu.sync_copy(dacc.at[pl.ds(base + t * ZR, ZR)],
                            out_h.at[c, s * (RPT // ZR) + t])

    return body(src4, dst4)


def _sc_edge_agg(p, src3, dst3, ew3):
    @functools.partial(
        pl.kernel,
        out_type=jax.ShapeDtypeStruct((NC, NS * (RPT // ZR), ZR, D),
                                      jnp.float32),
        mesh=_mesh(),
        scratch_types=[
            pltpu.VMEM((2, 1, B), jnp.int32),
            pltpu.VMEM((2, 1, B), jnp.int32),
            pltpu.VMEM((2, 1, B), jnp.float32),
            pltpu.VMEM((2, B, D), jnp.float32),
            pltpu.SemaphoreType.DMA,
            pltpu.SemaphoreType.DMA,
            pltpu.SemaphoreType.DMA,
            pltpu.SemaphoreType.DMA,
            pltpu.SemaphoreType.DMA,
            pltpu.SemaphoreType.DMA,
            pltpu.SemaphoreType.DMA,
            pltpu.SemaphoreType.DMA,
            pltpu.SemaphoreType.DMA,
            pltpu.SemaphoreType.DMA,
            pltpu.VMEM_SHARED((N, D), jnp.float32),
        ],
    )
    def body(p_h, src_h, dst_h, ew_h, out_h, sidx, didx, wbuf, rows,
             g0, g1, s0, s1, ps0, ps1, pw0, pw1, pd0, pd1, acc):
        c = lax.axis_index("c")
        s = lax.axis_index("s")
        w = c * NS + s
        base = s * RPT
        gsem = (g0, g1)
        ssem = (s0, s1)
        psrc = (ps0, ps1)
        pew = (pw0, pw1)
        pdst = (pd0, pd1)

        def fill_zero(i, carry):
            for j in range(D // LN):
                rows[0, i, pl.ds(j * LN, LN)] = jnp.zeros((LN,), jnp.float32)
            return carry

        lax.fori_loop(0, B, fill_zero, 0)
        for t in range(RPT // B):
            pltpu.sync_copy(rows.at[0], acc.at[pl.ds(base + t * B, B)])
        pltpu.sync_copy(rows.at[0, pl.ds(0, RPT % B)],
                        acc.at[pl.ds(base + (RPT // B) * B, RPT % B)])
        plsc.subcore_barrier()

        def scale(b):
            def grp(g, icarry):
                ewv = wbuf[b, 0, pl.ds(g * LN, LN)]
                for l in range(LN):
                    i = g * LN + l
                    coef = jnp.broadcast_to(ewv[l], (LN,))
                    for j in range(D // LN):
                        rows[b, i, pl.ds(j * LN, LN)] = (
                            rows[b, i, pl.ds(j * LN, LN)] * coef)
                return icarry

            lax.fori_loop(0, B // LN, grp, 0)

        def step(k, b):
            pltpu.make_async_copy(p_h.at[sidx.at[b, 0]], rows.at[b],
                                  gsem[b]).wait()

            @pl.when(k + 2 < NCHUNK)
            def _():
                pltpu.async_copy(src_h.at[w, k + 2], sidx.at[b], psrc[b])

            @pl.when(k >= 2)
            def _():
                pltpu.make_async_copy(ew_h.at[w, k], wbuf.at[b],
                                      pew[b]).wait()

            scale(b)

            @pl.when(k + 2 < NCHUNK)
            def _():
                pltpu.async_copy(ew_h.at[w, k + 2], wbuf.at[b], pew[b])

            @pl.when(k >= 2)
            def _():
                pltpu.make_async_copy(dst_h.at[w, k], didx.at[b],
                                      pdst[b]).wait()

            pltpu.async_copy(rows.at[b], acc.at[didx.at[b, 0]],
                             ssem[b], add=True).wait()

            @pl.when(k + 2 < NCHUNK)
            def _():
                pltpu.async_copy(dst_h.at[w, k + 2], didx.at[b], pdst[b])
                pltpu.make_async_copy(src_h.at[w, k + 2], sidx.at[b],
                                      psrc[b]).wait()
                pltpu.async_copy(p_h.at[sidx.at[b, 0]], rows.at[b], gsem[b])

        for b in (0, 1):
            pltpu.sync_copy(src_h.at[w, b], sidx.at[b])
            pltpu.sync_copy(dst_h.at[w, b], didx.at[b])
            pltpu.sync_copy(ew_h.at[w, b], wbuf.at[b])
            pltpu.async_copy(p_h.at[sidx.at[b, 0]], rows.at[b], gsem[b])

        def pair(i, carry):
            step(2 * i, 0)
            step(2 * i + 1, 1)
            return carry

        lax.fori_loop(0, NCHUNK // 2, pair, 0)
        if NCHUNK % 2:
            step(NCHUNK - 1, 0)

        plsc.subcore_barrier()
        for t in range(RPT // ZR):
            pltpu.sync_copy(acc.at[pl.ds(base + t * ZR, ZR)],
                            out_h.at[c, s * (RPT // ZR) + t])

    return body(p, src3, dst3, ew3)


def _tc_prep(degp, features):
    def body(deg_ref, x_ref, q_ref, nsd_ref):
        d = deg_ref[0] + deg_ref[1]
        od = d[:, 0:1]
        idg = d[:, 16:17]
        ns = lax.rsqrt(jnp.maximum(od, 1.0))
        nd = lax.rsqrt(jnp.maximum(idg, 1.0))
        q_ref[...] = x_ref[...] * ns
        nsd_ref[...] = jnp.concatenate(
            [ns, nd, jnp.zeros((N, 6), jnp.float32)], axis=1)

    return pl.pallas_call(
        body,
        out_shape=(
            jax.ShapeDtypeStruct((N, D), jnp.float32),
            jax.ShapeDtypeStruct((N, 8), jnp.float32),
        ),
    )(degp, features)


def _dot(a, b):
    return lax.dot_general(a, b, (((1,), (0,)), ((), ())),
                           precision=lax.Precision.HIGHEST,
                           preferred_element_type=jnp.float32)


def _tc_layer(aggp, nsd, W, gamma, beta, alpha, phi_w, phi_b, rho_w, rho_b):
    def body(aggp_ref, nsd_ref, w_ref, gamma_ref, beta_ref, alpha_ref,
             phiw_ref, phib_ref, rhow_ref, rhob_ref,
             q_ref, r_ref, m_ref):
        ns = nsd_ref[:, 0:1]
        nd = nsd_ref[:, 1:2]
        agg = (aggp_ref[0] + aggp_ref[1]) * nd
        z = _dot(agg, w_ref[...])
        alpha_v = alpha_ref[...]
        mu = jnp.mean(z, axis=0, keepdims=True)
        shifted = z - alpha_v * mu
        var = jnp.mean(shifted * shifted, axis=0, keepdims=True)
        hn = gamma_ref[...] * shifted * lax.rsqrt(var + EPS) + beta_ref[...]
        h = _leaky(hn)
        q_ref[...] = h * ns
        phi = _leaky(_dot(h, phiw_ref[...]) + phib_ref[...])
        pooled = jnp.mean(phi, axis=0, keepdims=True)
        r = _dot(pooled, rhow_ref[...]) + rhob_ref[...]
        r_ref[...] = _leaky(r)
        m_ref[...] = _leaky(jnp.mean(h, axis=0, keepdims=True))

    return pl.pallas_call(
        body,
        out_shape=(
            jax.ShapeDtypeStruct((N, H), jnp.float32),
            jax.ShapeDtypeStruct((1, RO), jnp.float32),
            jax.ShapeDtypeStruct((1, H), jnp.float32),
        ),
    )(aggp, nsd, W, gamma, beta, alpha, phi_w, phi_b, rho_w, rho_b)


def kernel(features, edge_index, edge_weights, W1, W2,
           gn1_gamma, gn1_beta, gn1_alpha, gn2_gamma, gn2_beta, gn2_alpha,
           ro1_phi_w, ro1_phi_b, ro1_rho_w, ro1_rho_b,
           ro2_phi_w, ro2_phi_b, ro2_rho_w, ro2_rho_b):
    src2 = edge_index[0].reshape(NW, NCHUNK, B)
    dst2 = edge_index[1].reshape(NW, NCHUNK, B)
    ew2 = edge_weights.reshape(NW, NCHUNK, B)
    src3 = src2.reshape(NW, NCHUNK, 1, B)
    dst3 = dst2.reshape(NW, NCHUNK, 1, B)
    ew3 = ew2.reshape(NW, NCHUNK, 1, B)
    src4 = src2.reshape(NW, NB, GD, B)
    dst4 = dst2.reshape(NW, NB, GD, B)

    degp = _sc_degrees(src4, dst4)
    degp = degp.reshape(NC, N, D)
    q1, nsd = _tc_prep(degp, features)

    g1g = gn1_gamma.reshape(1, H)
    g1b = gn1_beta.reshape(1, H)
    g1a = gn1_alpha.reshape(1, H)
    g2g = gn2_gamma.reshape(1, H)
    g2b = gn2_beta.reshape(1, H)
    g2a = gn2_alpha.reshape(1, H)

    aggp1 = _sc_edge_agg(q1, src3, dst3, ew3)
    aggp1 = aggp1.reshape(NC, N, D)
    q2, r1, m1 = _tc_layer(aggp1, nsd, W1, g1g, g1b, g1a,
                           ro1_phi_w, ro1_phi_b.reshape(1, H),
                           ro1_rho_w, ro1_rho_b.reshape(1, RO))
    aggp2 = _sc_edge_agg(q2, src3, dst3, ew3)
    aggp2 = aggp2.reshape(NC, N, D)
    _, r2, m2 = _tc_layer(aggp2, nsd, W2, g2g, g2b, g2a,
                          ro2_phi_w, ro2_phi_b.reshape(1, H),
                          ro2_rho_w, ro2_rho_b.reshape(1, RO))

    return jnp.concatenate([r1, m1, r2, m2], axis=1)


# speedup vs baseline: 6.6518x; 1.0073x over previous
import functools

import jax
import jax.numpy as jnp
from jax import lax
from jax.experimental import pallas as pl
from jax.experimental.pallas import tpu as pltpu
from jax.experimental.pallas import tpu_sc as plsc

N = 10000
E = 320000
D = 128
H = 128
RO = H // 2
EPS = 1e-05

NC = 2
NS = 16
LN = 16
NW = NC * NS
EW = E // NW
B = 80
NCHUNK = EW // B
RPT = N // NS
ZR = 25
ZB = 5


def _leaky(x):
    return jnp.where(x >= 0, x, 0.01 * x)


def _mesh():
    return plsc.VectorSubcoreMesh(core_axis_name="c", subcore_axis_name="s",
                                  num_cores=NC, num_subcores=NS)


GD = 25
NB = NCHUNK // GD


def _sc_degrees(src4, dst4):
    @functools.partial(
        pl.kernel,
        out_type=jax.ShapeDtypeStruct((NC, NS * (RPT // ZR), ZR, D),
                                      jnp.float32),
        mesh=_mesh(),
        scratch_types=[
            pltpu.VMEM((2, GD, B), jnp.int32),
            pltpu.VMEM((B, D), jnp.float32),
            pltpu.SemaphoreType.DMA,
            pltpu.SemaphoreType.DMA,
            pltpu.VMEM_SHARED((N, D), jnp.float32),
        ],
    )
    def body(src_h, dst_h, out_h, ibuf, ones_v, dsem, stsem, dacc):
        c = lax.axis_index("c")
        s = lax.axis_index("s")
        w = c * NS + s
        base = s * RPT

        def fill_zero(i, carry):
            for j in range(D // LN):
                ones_v[i, pl.ds(j * LN, LN)] = jnp.zeros((LN,), jnp.float32)
            return carry

        lax.fori_loop(0, B, fill_zero, 0)
        for t in range(RPT // B):
            pltpu.sync_copy(ones_v, dacc.at[pl.ds(base + t * B, B)])
        pltpu.sync_copy(ones_v.at[pl.ds(0, RPT % B)],
                        dacc.at[pl.ds(base + (RPT // B) * B, RPT % B)])

        for (which_h, lane0) in ((src_h, 0), (dst_h, LN)):
            def fill_ones(i, carry):
                ones_v[i, pl.ds(lane0, LN)] = jnp.full((LN,), 1.0,
                                                       jnp.float32)
                return carry

            lax.fori_loop(0, B, fill_ones, 0)
            if lane0 == 0:
                plsc.subcore_barrier()

            pltpu.sync_copy(which_h.at[w, 0], ibuf.at[0])

            def batch2(i2, carry):
                for bb in (0, 1):
                    g = 2 * i2 + bb

                    @pl.when(g + 1 < NB)
                    def _():
                        pltpu.async_copy(which_h.at[w, g + 1],
                                         ibuf.at[1 - bb], stsem)
                    for j0 in range(0, GD, 8):
                        hi = min(j0 + 8, GD)
                        for j in range(j0, hi):
                            pltpu.async_copy(ones_v,
                                             dacc.at[ibuf.at[bb, j]],
                                             dsem, add=True)
                        for j in range(j0, hi):
                            pltpu.make_async_copy(
                                ones_v, dacc.at[ibuf.at[bb, j]],
                                dsem).wait()

                    @pl.when(g + 1 < NB)
                    def _():
                        pltpu.make_async_copy(which_h.at[w, g + 1],
                                              ibuf.at[1 - bb], stsem).wait()
                return carry

            lax.fori_loop(0, NB // 2, batch2, 0)
            if NB % 2:
                bb = (NB - 1) % 2
                for j0 in range(0, GD, 8):
                    hi = min(j0 + 8, GD)
                    for j in range(j0, hi):
                        pltpu.async_copy(ones_v, dacc.at[ibuf.at[bb, j]],
                                         dsem, add=True)
                    for j in range(j0, hi):
                        pltpu.make_async_copy(ones_v,
                                              dacc.at[ibuf.at[bb, j]],
                                              dsem).wait()

            def fill_zero_blk(i, carry):
                ones_v[i, pl.ds(lane0, LN)] = jnp.zeros((LN,), jnp.float32)
                return carry

            if lane0 == 0:
                lax.fori_loop(0, B, fill_zero_blk, 0)

        plsc.subcore_barrier()
        for t in range(RPT // ZR):
            pltpu.sync_copy(dacc.at[pl.ds(base + t * ZR, ZR)],
                            out_h.at[c, s * (RPT // ZR) + t])

    return body(src4, dst4)


def _sc_edge_agg(p, src3, dst3, ew3):
    @functools.partial(
        pl.kernel,
        out_type=jax.ShapeDtypeStruct((NC, NS * (RPT // ZR), ZR, D),
                                      jnp.float32),
        mesh=_mesh(),
        scratch_types=[
            pltpu.VMEM((2, 1, B), jnp.int32),
            pltpu.VMEM((2, 1, B), jnp.int32),
            pltpu.VMEM((2, 1, B), jnp.float32),
            pltpu.VMEM((2, B, D), jnp.float32),
            pltpu.SemaphoreType.DMA,
            pltpu.SemaphoreType.DMA,
            pltpu.SemaphoreType.DMA,
            pltpu.SemaphoreType.DMA,
            pltpu.SemaphoreType.DMA,
            pltpu.SemaphoreType.DMA,
            pltpu.SemaphoreType.DMA,
            pltpu.SemaphoreType.DMA,
            pltpu.SemaphoreType.DMA,
            pltpu.SemaphoreType.DMA,
            pltpu.VMEM_SHARED((N, D), jnp.float32),
        ],
    )
    def body(p_h, src_h, dst_h, ew_h, out_h, sidx, didx, wbuf, rows,
             g0, g1, s0, s1, ps0, ps1, pw0, pw1, pd0, pd1, acc):
        c = lax.axis_index("c")
        s = lax.axis_index("s")
        w = c * NS + s
        base = s * RPT
        gsem = (g0, g1)
        ssem = (s0, s1)
        psrc = (ps0, ps1)
        pew = (pw0, pw1)
        pdst = (pd0, pd1)

        def fill_zero(i, carry):
            for j in range(D // LN):
                rows[0, i, pl.ds(j * LN, LN)] = jnp.zeros((LN,), jnp.float32)
            return carry

        lax.fori_loop(0, B, fill_zero, 0)
        for t in range(RPT // B):
            pltpu.sync_copy(rows.at[0], acc.at[pl.ds(base + t * B, B)])
        pltpu.sync_copy(rows.at[0, pl.ds(0, RPT % B)],
                        acc.at[pl.ds(base + (RPT // B) * B, RPT % B)])
        plsc.subcore_barrier()

        def scale(b):
            def grp(g, icarry):
                ewv = wbuf[b, 0, pl.ds(g * LN, LN)]
                for l in range(LN):
                    i = g * LN + l
                    coef = jnp.broadcast_to(ewv[l], (LN,))
                    for j in range(D // LN):
                        rows[b, i, pl.ds(j * LN, LN)] = (
                            rows[b, i, pl.ds(j * LN, LN)] * coef)
                return icarry

            lax.fori_loop(0, B // LN, grp, 0)

        def step(k, b):
            pltpu.make_async_copy(p_h.at[sidx.at[b, 0]], rows.at[b],
                                  gsem[b]).wait()

            @pl.when(k + 2 < NCHUNK)
            def _():
                pltpu.async_copy(src_h.at[w, k + 2], sidx.at[b], psrc[b])

            @pl.when(k >= 2)
            def _():
                pltpu.make_async_copy(ew_h.at[w, k], wbuf.at[b],
                                      pew[b]).wait()

            scale(b)

            @pl.when(k + 2 < NCHUNK)
            def _():
                pltpu.async_copy(ew_h.at[w, k + 2], wbuf.at[b], pew[b])

            @pl.when(k >= 2)
            def _():
                pltpu.make_async_copy(dst_h.at[w, k], didx.at[b],
                                      pdst[b]).wait()

            pltpu.async_copy(rows.at[b], acc.at[didx.at[b, 0]],
                             ssem[b], add=True).wait()

            @pl.when(k + 2 < NCHUNK)
            def _():
                pltpu.async_copy(dst_h.at[w, k + 2], didx.at[b], pdst[b])
                pltpu.make_async_copy(src_h.at[w, k + 2], sidx.at[b],
                                      psrc[b]).wait()
                pltpu.async_copy(p_h.at[sidx.at[b, 0]], rows.at[b], gsem[b])

        for b in (0, 1):
            pltpu.sync_copy(src_h.at[w, b], sidx.at[b])
            pltpu.sync_copy(dst_h.at[w, b], didx.at[b])
            pltpu.sync_copy(ew_h.at[w, b], wbuf.at[b])
            pltpu.async_copy(p_h.at[sidx.at[b, 0]], rows.at[b], gsem[b])

        def pair(i, carry):
            step(2 * i, 0)
            step(2 * i + 1, 1)
            return carry

        lax.fori_loop(0, NCHUNK // 2, pair, 0)
        if NCHUNK % 2:
            step(NCHUNK - 1, 0)

        plsc.subcore_barrier()
        for t in range(RPT // ZR):
            pltpu.sync_copy(acc.at[pl.ds(base + t * ZR, ZR)],
                            out_h.at[c, s * (RPT // ZR) + t])

    return body(p, src3, dst3, ew3)


def _tc_prep(degp, features):
    def body(deg_ref, x_ref, q_ref, nsd_ref):
        d = deg_ref[0] + deg_ref[1]
        od = d[:, 0:1]
        idg = d[:, 16:17]
        ns = lax.rsqrt(jnp.maximum(od, 1.0))
        nd = lax.rsqrt(jnp.maximum(idg, 1.0))
        q_ref[...] = x_ref[...] * ns
        nsd_ref[...] = jnp.concatenate(
            [ns, nd, jnp.zeros((N, 6), jnp.float32)], axis=1)

    return pl.pallas_call(
        body,
        out_shape=(
            jax.ShapeDtypeStruct((N, D), jnp.float32),
            jax.ShapeDtypeStruct((N, 8), jnp.float32),
        ),
    )(degp, features)


def _dot(a, b):
    return lax.dot_general(a, b, (((1,), (0,)), ((), ())),
                           precision=lax.Precision.HIGHEST,
                           preferred_element_type=jnp.float32)


def _tc_layer(aggp, nsd, W, gamma, beta, alpha, phi_w, phi_b, rho_w, rho_b):
    def body(aggp_ref, nsd_ref, w_ref, gamma_ref, beta_ref, alpha_ref,
             phiw_ref, phib_ref, rhow_ref, rhob_ref,
             q_ref, r_ref, m_ref):
        ns = nsd_ref[:, 0:1]
        nd = nsd_ref[:, 1:2]
        agg = (aggp_ref[0] + aggp_ref[1]) * nd
        z = _dot(agg, w_ref[...])
        alpha_v = alpha_ref[...]
        mu = jnp.mean(z, axis=0, keepdims=True)
        shifted = z - alpha_v * mu
        var = jnp.mean(shifted * shifted, axis=0, keepdims=True)
        hn = gamma_ref[...] * shifted * lax.rsqrt(var + EPS) + beta_ref[...]
        h = _leaky(hn)
        q_ref[...] = h * ns
        phi = _leaky(_dot(h, phiw_ref[...]) + phib_ref[...])
        pooled = jnp.mean(phi, axis=0, keepdims=True)
        r = _dot(pooled, rhow_ref[...]) + rhob_ref[...]
        r_ref[...] = _leaky(r)
        m_ref[...] = _leaky(jnp.mean(h, axis=0, keepdims=True))

    return pl.pallas_call(
        body,
        out_shape=(
            jax.ShapeDtypeStruct((N, H), jnp.float32),
            jax.ShapeDtypeStruct((1, RO), jnp.float32),
            jax.ShapeDtypeStruct((1, H), jnp.float32),
        ),
    )(aggp, nsd, W, gamma, beta, alpha, phi_w, phi_b, rho_w, rho_b)


def kernel(features, edge_index, edge_weights, W1, W2,
           gn1_gamma, gn1_beta, gn1_alpha, gn2_gamma, gn2_beta, gn2_alpha,
           ro1_phi_w, ro1_phi_b, ro1_rho_w, ro1_rho_b,
           ro2_phi_w, ro2_phi_b, ro2_rho_w, ro2_rho_b):
    src2 = edge_index[0].reshape(NW, NCHUNK, B)
    dst2 = edge_index[1].reshape(NW, NCHUNK, B)
    ew2 = edge_weights.reshape(NW, NCHUNK, B)
    src3 = src2.reshape(NW, NCHUNK, 1, B)
    dst3 = dst2.reshape(NW, NCHUNK, 1, B)
    ew3 = ew2.reshape(NW, NCHUNK, 1, B)
    src4 = src2.reshape(NW, NB, GD, B)
    dst4 = dst2.reshape(NW, NB, GD, B)

    degp = _sc_degrees(src4, dst4)
    degp = degp.reshape(NC, N, D)
    q1, nsd = _tc_prep(degp, features)

    g1g = gn1_gamma.reshape(1, H)
    g1b = gn1_beta.reshape(1, H)
    g1a = gn1_alpha.reshape(1, H)
    g2g = gn2_gamma.reshape(1, H)
    g2b = gn2_beta.reshape(1, H)
    g2a = gn2_alpha.reshape(1, H)

    aggp1 = _sc_edge_agg(q1, src3, dst3, ew3)
    aggp1 = aggp1.reshape(NC, N, D)
    q2, r1, m1 = _tc_layer(aggp1, nsd, W1, g1g, g1b, g1a,
                           ro1_phi_w, ro1_phi_b.reshape(1, H),
                           ro1_rho_w, ro1_rho_b.reshape(1, RO))
    aggp2 = _sc_edge_agg(q2, src3, dst3, ew3)
    aggp2 = aggp2.reshape(NC, N, D)
    _, r2, m2 = _tc_layer(aggp2, nsd, W2, g2g, g2b, g2a,
                          ro2_phi_w, ro2_phi_b.reshape(1, H),
                          ro2_rho_w, ro2_rho_b.reshape(1, RO))

    return jnp.concatenate([r1, m1, r2, m2], axis=1)


# speedup vs baseline: 6.7634x; 1.0168x over previous
import functools

import jax
import jax.numpy as jnp
from jax import lax
from jax.experimental import pallas as pl
from jax.experimental.pallas import tpu as pltpu
from jax.experimental.pallas import tpu_sc as plsc

N = 10000
E = 320000
D = 128
H = 128
RO = H // 2
EPS = 1e-05

NC = 2
NS = 16
LN = 16
NW = NC * NS
EW = E // NW
B = 80
NCHUNK = EW // B
RPT = N // NS
ZR = 25
ZB = 5


def _leaky(x):
    return jnp.where(x >= 0, x, 0.01 * x)


def _mesh():
    return plsc.VectorSubcoreMesh(core_axis_name="c", subcore_axis_name="s",
                                  num_cores=NC, num_subcores=NS)


GD = 25
NB = NCHUNK // GD


def _sc_degrees(src4, dst4):
    @functools.partial(
        pl.kernel,
        out_type=jax.ShapeDtypeStruct((NC, NS * (RPT // ZR), ZR, D),
                                      jnp.float32),
        mesh=_mesh(),
        scratch_types=[
            pltpu.VMEM((2, GD, B), jnp.int32),
            pltpu.VMEM((B, D), jnp.float32),
            pltpu.SemaphoreType.DMA,
            pltpu.SemaphoreType.DMA,
            pltpu.VMEM_SHARED((N, D), jnp.float32),
        ],
    )
    def body(src_h, dst_h, out_h, ibuf, ones_v, dsem, stsem, dacc):
        c = lax.axis_index("c")
        s = lax.axis_index("s")
        w = c * NS + s
        base = s * RPT

        def fill_zero(i, carry):
            for j in range(D // LN):
                ones_v[i, pl.ds(j * LN, LN)] = jnp.zeros((LN,), jnp.float32)
            return carry

        lax.fori_loop(0, B, fill_zero, 0)
        for t in range(RPT // B):
            pltpu.sync_copy(ones_v, dacc.at[pl.ds(base + t * B, B)])
        pltpu.sync_copy(ones_v.at[pl.ds(0, RPT % B)],
                        dacc.at[pl.ds(base + (RPT // B) * B, RPT % B)])

        for (which_h, lane0) in ((src_h, 0), (dst_h, LN)):
            def fill_ones(i, carry):
                ones_v[i, pl.ds(lane0, LN)] = jnp.full((LN,), 1.0,
                                                       jnp.float32)
                return carry

            lax.fori_loop(0, B, fill_ones, 0)
            if lane0 == 0:
                plsc.subcore_barrier()

            pltpu.sync_copy(which_h.at[w, 0], ibuf.at[0])

            def batch2(i2, carry):
                for bb in (0, 1):
                    g = 2 * i2 + bb

                    @pl.when(g + 1 < NB)
                    def _():
                        pltpu.async_copy(which_h.at[w, g + 1],
                                         ibuf.at[1 - bb], stsem)
                    for j0 in range(0, GD, 8):
                        hi = min(j0 + 8, GD)
                        for j in range(j0, hi):
                            pltpu.async_copy(ones_v,
                                             dacc.at[ibuf.at[bb, j]],
                                             dsem, add=True)
                        for j in range(j0, hi):
                            pltpu.make_async_copy(
                                ones_v, dacc.at[ibuf.at[bb, j]],
                                dsem).wait()

                    @pl.when(g + 1 < NB)
                    def _():
                        pltpu.make_async_copy(which_h.at[w, g + 1],
                                              ibuf.at[1 - bb], stsem).wait()
                return carry

            lax.fori_loop(0, NB // 2, batch2, 0)
            if NB % 2:
                bb = (NB - 1) % 2
                for j0 in range(0, GD, 8):
                    hi = min(j0 + 8, GD)
                    for j in range(j0, hi):
                        pltpu.async_copy(ones_v, dacc.at[ibuf.at[bb, j]],
                                         dsem, add=True)
                    for j in range(j0, hi):
                        pltpu.make_async_copy(ones_v,
                                              dacc.at[ibuf.at[bb, j]],
                                              dsem).wait()

            def fill_zero_blk(i, carry):
                ones_v[i, pl.ds(lane0, LN)] = jnp.zeros((LN,), jnp.float32)
                return carry

            if lane0 == 0:
                lax.fori_loop(0, B, fill_zero_blk, 0)

        plsc.subcore_barrier()
        for t in range(RPT // ZR):
            pltpu.sync_copy(dacc.at[pl.ds(base + t * ZR, ZR)],
                            out_h.at[c, s * (RPT // ZR) + t])

    return body(src4, dst4)


HA = 32
HB = B - HA


def _sc_edge_agg(p, src3, dstA, dstB, ew3):
    @functools.partial(
        pl.kernel,
        out_type=jax.ShapeDtypeStruct((NC, NS * (RPT // ZR), ZR, D),
                                      jnp.float32),
        mesh=_mesh(),
        scratch_types=[
            pltpu.VMEM((2, 1, B), jnp.int32),
            pltpu.VMEM((2, 1, HA), jnp.int32),
            pltpu.VMEM((2, 1, HB), jnp.int32),
            pltpu.VMEM((2, 1, B), jnp.float32),
            pltpu.VMEM((2, B, D), jnp.float32),
            pltpu.SemaphoreType.DMA,
            pltpu.SemaphoreType.DMA,
            pltpu.SemaphoreType.DMA,
            pltpu.SemaphoreType.DMA,
            pltpu.SemaphoreType.DMA,
            pltpu.SemaphoreType.DMA,
            pltpu.SemaphoreType.DMA,
            pltpu.SemaphoreType.DMA,
            pltpu.SemaphoreType.DMA,
            pltpu.SemaphoreType.DMA,
            pltpu.VMEM_SHARED((N, D), jnp.float32),
        ],
    )
    def body(p_h, src_h, dstA_h, dstB_h, ew_h, out_h, sidx, didxA, didxB,
             wbuf, rows, g0, g1, s0, s1, ps0, ps1, pw0, pw1, pd0, pd1, acc):
        c = lax.axis_index("c")
        s = lax.axis_index("s")
        w = c * NS + s
        base = s * RPT
        gsem = (g0, g1)
        ssem = (s0, s1)
        psrc = (ps0, ps1)
        pew = (pw0, pw1)
        pdst = (pd0, pd1)

        def fill_zero(i, carry):
            for j in range(D // LN):
                rows[0, i, pl.ds(j * LN, LN)] = jnp.zeros((LN,), jnp.float32)
            return carry

        lax.fori_loop(0, B, fill_zero, 0)
        for t in range(RPT // B):
            pltpu.sync_copy(rows.at[0], acc.at[pl.ds(base + t * B, B)])
        pltpu.sync_copy(rows.at[0, pl.ds(0, RPT % B)],
                        acc.at[pl.ds(base + (RPT // B) * B, RPT % B)])
        plsc.subcore_barrier()

        def scale(b, g0_, g1_):
            def grp(g, icarry):
                ewv = wbuf[b, 0, pl.ds(g * LN, LN)]
                for l in range(LN):
                    i = g * LN + l
                    coef = jnp.broadcast_to(ewv[l], (LN,))
                    for j in range(D // LN):
                        rows[b, i, pl.ds(j * LN, LN)] = (
                            rows[b, i, pl.ds(j * LN, LN)] * coef)
                return icarry

            lax.fori_loop(g0_, g1_, grp, 0)

        def step(k, b):
            pltpu.make_async_copy(p_h.at[sidx.at[b, 0]], rows.at[b],
                                  gsem[b]).wait()

            @pl.when(k + 2 < NCHUNK)
            def _():
                pltpu.async_copy(src_h.at[w, k + 2], sidx.at[b], psrc[b])

            @pl.when(k >= 2)
            def _():
                pltpu.make_async_copy(ew_h.at[w, k], wbuf.at[b],
                                      pew[b]).wait()
                pltpu.make_async_copy(dstA_h.at[w, k], didxA.at[b],
                                      pdst[b]).wait()
                pltpu.make_async_copy(dstB_h.at[w, k], didxB.at[b],
                                      pdst[b]).wait()

            scale(b, 0, HA // LN)
            scA = pltpu.async_copy(rows.at[b, pl.ds(0, HA)],
                                   acc.at[didxA.at[b, 0]], ssem[b],
                                   add=True)
            scale(b, HA // LN, B // LN)
            scB = pltpu.async_copy(rows.at[b, pl.ds(HA, HB)],
                                   acc.at[didxB.at[b, 0]], ssem[b],
                                   add=True)

            @pl.when(k + 2 < NCHUNK)
            def _():
                pltpu.async_copy(ew_h.at[w, k + 2], wbuf.at[b], pew[b])

            scA.wait()
            scB.wait()

            @pl.when(k + 2 < NCHUNK)
            def _():
                pltpu.async_copy(dstA_h.at[w, k + 2], didxA.at[b], pdst[b])
                pltpu.async_copy(dstB_h.at[w, k + 2], didxB.at[b], pdst[b])
                pltpu.make_async_copy(src_h.at[w, k + 2], sidx.at[b],
                                      psrc[b]).wait()
                pltpu.async_copy(p_h.at[sidx.at[b, 0]], rows.at[b], gsem[b])

        for b in (0, 1):
            pltpu.sync_copy(src_h.at[w, b], sidx.at[b])
            pltpu.sync_copy(dstA_h.at[w, b], didxA.at[b])
            pltpu.sync_copy(dstB_h.at[w, b], didxB.at[b])
            pltpu.sync_copy(ew_h.at[w, b], wbuf.at[b])
            pltpu.async_copy(p_h.at[sidx.at[b, 0]], rows.at[b], gsem[b])

        def pair(i, carry):
            step(2 * i, 0)
            step(2 * i + 1, 1)
            return carry

        lax.fori_loop(0, NCHUNK // 2, pair, 0)
        if NCHUNK % 2:
            step(NCHUNK - 1, 0)

        plsc.subcore_barrier()
        for t in range(RPT // ZR):
            pltpu.sync_copy(acc.at[pl.ds(base + t * ZR, ZR)],
                            out_h.at[c, s * (RPT // ZR) + t])

    return body(p, src3, dstA, dstB, ew3)


def _tc_prep(degp, features):
    def body(deg_ref, x_ref, q_ref, nsd_ref):
        d = deg_ref[0] + deg_ref[1]
        od = d[:, 0:1]
        idg = d[:, 16:17]
        ns = lax.rsqrt(jnp.maximum(od, 1.0))
        nd = lax.rsqrt(jnp.maximum(idg, 1.0))
        q_ref[...] = x_ref[...] * ns
        nsd_ref[...] = jnp.concatenate(
            [ns, nd, jnp.zeros((N, 6), jnp.float32)], axis=1)

    return pl.pallas_call(
        body,
        out_shape=(
            jax.ShapeDtypeStruct((N, D), jnp.float32),
            jax.ShapeDtypeStruct((N, 8), jnp.float32),
        ),
    )(degp, features)


def _dot(a, b):
    return lax.dot_general(a, b, (((1,), (0,)), ((), ())),
                           precision=lax.Precision.HIGHEST,
                           preferred_element_type=jnp.float32)


def _tc_layer(aggp, nsd, W, gamma, beta, alpha, phi_w, phi_b, rho_w, rho_b):
    def body(aggp_ref, nsd_ref, w_ref, gamma_ref, beta_ref, alpha_ref,
             phiw_ref, phib_ref, rhow_ref, rhob_ref,
             q_ref, r_ref, m_ref):
        ns = nsd_ref[:, 0:1]
        nd = nsd_ref[:, 1:2]
        agg = (aggp_ref[0] + aggp_ref[1]) * nd
        z = _dot(agg, w_ref[...])
        alpha_v = alpha_ref[...]
        mu = jnp.mean(z, axis=0, keepdims=True)
        shifted = z - alpha_v * mu
        var = jnp.mean(shifted * shifted, axis=0, keepdims=True)
        hn = gamma_ref[...] * shifted * lax.rsqrt(var + EPS) + beta_ref[...]
        h = _leaky(hn)
        q_ref[...] = h * ns
        phi = _leaky(_dot(h, phiw_ref[...]) + phib_ref[...])
        pooled = jnp.mean(phi, axis=0, keepdims=True)
        r = _dot(pooled, rhow_ref[...]) + rhob_ref[...]
        r_ref[...] = _leaky(r)
        m_ref[...] = _leaky(jnp.mean(h, axis=0, keepdims=True))

    return pl.pallas_call(
        body,
        out_shape=(
            jax.ShapeDtypeStruct((N, H), jnp.float32),
            jax.ShapeDtypeStruct((1, RO), jnp.float32),
            jax.ShapeDtypeStruct((1, H), jnp.float32),
        ),
    )(aggp, nsd, W, gamma, beta, alpha, phi_w, phi_b, rho_w, rho_b)


def kernel(features, edge_index, edge_weights, W1, W2,
           gn1_gamma, gn1_beta, gn1_alpha, gn2_gamma, gn2_beta, gn2_alpha,
           ro1_phi_w, ro1_phi_b, ro1_rho_w, ro1_rho_b,
           ro2_phi_w, ro2_phi_b, ro2_rho_w, ro2_rho_b):
    src2 = edge_index[0].reshape(NW, NCHUNK, B)
    dst2 = edge_index[1].reshape(NW, NCHUNK, B)
    ew2 = edge_weights.reshape(NW, NCHUNK, B)
    src3 = src2.reshape(NW, NCHUNK, 1, B)
    ew3 = ew2.reshape(NW, NCHUNK, 1, B)
    dstA = dst2[:, :, :HA].reshape(NW, NCHUNK, 1, HA)
    dstB = dst2[:, :, HA:].reshape(NW, NCHUNK, 1, HB)
    src4 = src2.reshape(NW, NB, GD, B)
    dst4 = dst2.reshape(NW, NB, GD, B)

    degp = _sc_degrees(src4, dst4)
    degp = degp.reshape(NC, N, D)
    q1, nsd = _tc_prep(degp, features)

    g1g = gn1_gamma.reshape(1, H)
    g1b = gn1_beta.reshape(1, H)
    g1a = gn1_alpha.reshape(1, H)
    g2g = gn2_gamma.reshape(1, H)
    g2b = gn2_beta.reshape(1, H)
    g2a = gn2_alpha.reshape(1, H)

    aggp1 = _sc_edge_agg(q1, src3, dstA, dstB, ew3)
    aggp1 = aggp1.reshape(NC, N, D)
    q2, r1, m1 = _tc_layer(aggp1, nsd, W1, g1g, g1b, g1a,
                           ro1_phi_w, ro1_phi_b.reshape(1, H),
                           ro1_rho_w, ro1_rho_b.reshape(1, RO))
    aggp2 = _sc_edge_agg(q2, src3, dstA, dstB, ew3)
    aggp2 = aggp2.reshape(NC, N, D)
    _, r2, m2 = _tc_layer(aggp2, nsd, W2, g2g, g2b, g2a,
                          ro2_phi_w, ro2_phi_b.reshape(1, H),
                          ro2_rho_w, ro2_rho_b.reshape(1, RO))

    return jnp.concatenate([r1, m1, r2, m2], axis=1)
